# Initial kernel scaffold; baseline (speedup 1.0000x reference)
import jax
import jax.numpy as jnp
from jax.experimental import pallas as pl


def kernel(x1, x2, rbf0, sbf, t, idx_kj, idx_ji, W_rbf1, W_rbf2, W_sbf1, W_sbf2, W_t1, W_t2, W_rbf, W_kj, b_kj, W_ji, b_ji, W_down, W_up, rb_W1, rb_b1, rb_W2, rb_b2, W_lin, b_lin, ra0_W1, ra0_b1, ra0_W2, ra0_b2, ra1_W1, ra1_b1, ra1_W2, ra1_b2):
    raise NotImplementedError("write your pallas kernel here")


# speedup vs baseline: 1.0539x; 1.0539x over previous
import functools

import jax
import jax.numpy as jnp
from jax import lax
from jax.experimental import pallas as pl
from jax.experimental.pallas import tpu as pltpu
from jax.experimental.pallas import tpu_sc as plsc

E = 160000
T = 640000
HC = 128
INT = 64

NCORE = 2
NSUB = 16
NRANGE = 14
RANGE = 11520
EPAD = NRANGE * RANGE
BLK = 2560
NBLOCK = T // BLK
NCH = BLK // 16
NB = 128
TRASH0 = BLK + NB
CAP = 2752
ENCW = (BLK // 128 + 1) * 128
SUBROWS = RANGE // NSUB

BEA = 4000
BTB = 8000


def _silu(x):
    return x * jax.nn.sigmoid(x)


def _tca_body(x1_ref, rbf0_ref, Wji_ref, bji_ref, Wkj_ref, bkj_ref,
              Wr1_ref, Wr2_ref, Wd_ref, xji_ref, xkd_ref):
    x1 = x1_ref[...]
    xji_ref[...] = _silu(x1 @ Wji_ref[...] + bji_ref[...])
    rbf = (rbf0_ref[...] @ Wr1_ref[...]) @ Wr2_ref[...]
    xk = _silu(x1 @ Wkj_ref[...] + bkj_ref[...]) * rbf
    xkd = _silu(xk @ Wd_ref[...])
    xkd_ref[...] = jnp.concatenate(
        [xkd, jnp.zeros((xkd.shape[0], HC - INT), xkd.dtype)], axis=1)


def _tcb_body(sbf_ref, t_ref, Ws1_ref, Ws2_ref, Wt1_ref, Wt2_ref, m_ref):
    A = Ws1_ref[...] @ Ws2_ref[...]
    B = Wt1_ref[...] @ Wt2_ref[...]
    mm = (sbf_ref[...] @ A) * (t_ref[...] @ B)
    m_ref[...] = jnp.concatenate(
        [mm, jnp.zeros((mm.shape[0], HC - INT), mm.dtype)], axis=1)


def _tcp_body(ji_ref, enc_ref):
    jib = ji_ref[0]
    f32 = jnp.float32
    rows = BLK // 128
    lane16 = lax.broadcasted_iota(jnp.int32, (rows, 128), 1) % 16
    U = (lax.broadcasted_iota(jnp.int32, (128, 128), 0)
         <= lax.broadcasted_iota(jnp.int32, (128, 128), 1)).astype(f32)
    Ls = (lax.broadcasted_iota(jnp.int32, (rows, rows), 1)
          < lax.broadcasted_iota(jnp.int32, (rows, rows), 0)).astype(f32)
    for r in range(NRANGE):
        lo = r * RANGE
        mk = (jib >= lo) & (jib < lo + RANGE)
        mf = mk.astype(f32)
        p = jax.lax.dot(mf, U, preferred_element_type=f32)
        offs = jax.lax.dot(Ls, p, preferred_element_type=f32)[:, 127:128]
        pos = p + offs
        enc = jnp.where(mk, pos.astype(jnp.int32) - 1, TRASH0 + lane16)
        cnt = pos[rows - 1:rows, 127:128]
        cnt_row = jnp.broadcast_to(cnt, (1, 128)).astype(jnp.int32)
        enc_ref[0, r] = jnp.concatenate([enc, cnt_row], axis=0)


def _tcc_body(y_ref, xji_ref, x1_ref, rbf0_ref, Wup_ref,
              rbW1_ref, rbb1_ref, rbW2_ref, rbb2_ref,
              Wlin_ref, blin_ref,
              a0W1_ref, a0b1_ref, a0W2_ref, a0b2_ref,
              a1W1_ref, a1b1_ref, a1W2_ref, a1b2_ref,
              Wrbf_ref, e1_ref, e2_ref):
    e1 = xji_ref[...] + _silu(y_ref[...] @ Wup_ref[...])
    e1 = e1 + _silu(_silu(e1 @ rbW1_ref[...] + rbb1_ref[...])
                    @ rbW2_ref[...] + rbb2_ref[...])
    e1 = _silu(e1 @ Wlin_ref[...] + blin_ref[...]) + x1_ref[...]
    e1 = e1 + _silu(_silu(e1 @ a0W1_ref[...] + a0b1_ref[...])
                    @ a0W2_ref[...] + a0b2_ref[...])
    e1 = e1 + _silu(_silu(e1 @ a1W1_ref[...] + a1b1_ref[...])
                    @ a1W2_ref[...] + a1b2_ref[...])
    e1_ref[...] = e1
    e2_ref[...] = (rbf0_ref[...] @ Wrbf_ref[...]) * e1


def _sc_body(tbl, m, kj, ji, enc, zrows, y,
             acc, sp_tid, sp_kj, sp_d,
             ji_v, kj_v, enc_v, pos_v, tids_v,
             tid_loc, kj_loc, d_loc, dstb_v, mrows_v, trows_v, prod_v,
             sem_a, sem_b):
    cid = lax.axis_index("c")
    sid = lax.axis_index("s")
    row0 = sid * SUBROWS
    lanes = lax.iota(jnp.int32, 16)
    sid_cap = sid * CAP

    pltpu.sync_copy(zrows, acc.at[pl.ds(row0, SUBROWS)])
    plsc.subcore_barrier()

    nblk = jnp.where(sid < NBLOCK - (NBLOCK // NSUB) * NSUB,
                     NBLOCK // NSUB + 1, NBLOCK // NSUB)

    for ri in range(NRANGE // NCORE):
        r = cid * (NRANGE // NCORE) + ri
        lo = r * RANGE

        def block_body(k, _, r=r, lo=lo):
            b = sid + k * NSUB
            tbase = b * BLK
            pltpu.sync_copy(ji.at[pl.ds(tbase, BLK)], ji_v)
            pltpu.sync_copy(kj.at[pl.ds(tbase, BLK)], kj_v)
            pltpu.sync_copy(enc.at[pl.ds((b * NRANGE + r) * ENCW, ENCW)],
                            enc_v)

            def chunk_body(i, _):
                o = i * 16
                pos_v[pl.ds(o, 16)] = enc_v[pl.ds(o, 16)] + sid_cap
                ji_v[pl.ds(o, 16)] = ji_v[pl.ds(o, 16)] - lo
                tids_v[pl.ds(o, 16)] = (tbase + o) + lanes
                return 0

            lax.fori_loop(0, NCH, chunk_body, 0)

            pltpu.sync_copy(tids_v, sp_tid.at[pos_v])
            pltpu.sync_copy(kj_v, sp_kj.at[pos_v])
            pltpu.sync_copy(ji_v, sp_d.at[pos_v])
            pltpu.sync_copy(sp_tid.at[pl.ds(sid_cap, CAP)], tid_loc)
            pltpu.sync_copy(sp_kj.at[pl.ds(sid_cap, CAP)], kj_loc)
            pltpu.sync_copy(sp_d.at[pl.ds(sid_cap, CAP)], d_loc)

            cnt = enc_v[pl.ds(BLK, 16)][0]

            for k2 in range(NB // 16):
                tid_loc[pl.ds(cnt + k2 * 16, 16)] = jnp.zeros((16,),
                                                              jnp.int32)
                kj_loc[pl.ds(cnt + k2 * 16, 16)] = jnp.zeros((16,), jnp.int32)
                d_loc[pl.ds(cnt + k2 * 16, 16)] = jnp.full((16,), RANGE,
                                                           jnp.int32)
            nb = (cnt + (NB - 1)) >> 7

            def batch_body(bb, _):
                off = bb * NB
                for k3 in range(NB // 16):
                    dstb_v[pl.ds(k3 * 16, 16)] = d_loc[pl.ds(off + k3 * 16,
                                                             16)]
                cp_m = pltpu.async_copy(m.at[tid_loc.at[pl.ds(off, NB)]],
                                        mrows_v, sem_a)
                cp_t = pltpu.async_copy(tbl.at[kj_loc.at[pl.ds(off, NB)]],
                                        trows_v, sem_b)
                cp_m.wait()
                cp_t.wait()

                def mul_row(i2, _):
                    for jj in range(INT // 16):
                        tc = trows_v[i2, pl.ds(jj * 16, 16)]
                        mc = mrows_v[i2, pl.ds(jj * 16, 16)]
                        prod_v[i2, pl.ds(jj * 16, 16)] = tc * mc
                    return 0

                lax.fori_loop(0, NB, mul_row, 0)
                pltpu.sync_copy(prod_v, acc.at[dstb_v], add=True)
                return 0

            lax.fori_loop(0, nb, batch_body, 0)
            return 0

        lax.fori_loop(0, nblk, block_body, 0)

        plsc.subcore_barrier()
        pltpu.sync_copy(acc.at[pl.ds(row0, SUBROWS)],
                        y.at[pl.ds(lo + row0, SUBROWS)])
        if ri != NRANGE // NCORE - 1:
            pltpu.sync_copy(zrows, acc.at[pl.ds(row0, SUBROWS)])
        plsc.subcore_barrier()


_sc_scatter = functools.partial(
    pl.kernel,
    out_type=jax.ShapeDtypeStruct((EPAD, INT), jnp.float32),
    mesh=plsc.VectorSubcoreMesh(core_axis_name="c", subcore_axis_name="s"),
    scratch_types=[
        pltpu.VMEM_SHARED((RANGE + 8, INT), jnp.float32),
        pltpu.VMEM_SHARED((NSUB * CAP,), jnp.int32),
        pltpu.VMEM_SHARED((NSUB * CAP,), jnp.int32),
        pltpu.VMEM_SHARED((NSUB * CAP,), jnp.int32),
        pltpu.VMEM((BLK,), jnp.int32),
        pltpu.VMEM((BLK,), jnp.int32),
        pltpu.VMEM((ENCW,), jnp.int32),
        pltpu.VMEM((BLK,), jnp.int32),
        pltpu.VMEM((BLK,), jnp.int32),
        pltpu.VMEM((CAP,), jnp.int32),
        pltpu.VMEM((CAP,), jnp.int32),
        pltpu.VMEM((CAP,), jnp.int32),
        pltpu.VMEM((NB,), jnp.int32),
        pltpu.VMEM((NB, HC), jnp.float32),
        pltpu.VMEM((NB, HC), jnp.float32),
        pltpu.VMEM((NB, INT), jnp.float32),
        pltpu.SemaphoreType.DMA,
        pltpu.SemaphoreType.DMA,
    ],
)(_sc_body)


def _rep(shape):
    nd = len(shape)
    return pl.BlockSpec(shape, lambda i, _nd=nd: (0,) * _nd)


def kernel(x1, x2, rbf0, sbf, t, idx_kj, idx_ji,
           W_rbf1, W_rbf2, W_sbf1, W_sbf2, W_t1, W_t2, W_rbf,
           W_kj, b_kj, W_ji, b_ji, W_down, W_up,
           rb_W1, rb_b1, rb_W2, rb_b2,
           W_lin, b_lin,
           ra0_W1, ra0_b1, ra0_W2, ra0_b2,
           ra1_W1, ra1_b1, ra1_W2, ra1_b2):
    f32 = jnp.float32
    idx_kj = idx_kj.astype(jnp.int32)
    idx_ji = idx_ji.astype(jnp.int32)
    b_kj2 = b_kj.reshape(1, HC)
    b_ji2 = b_ji.reshape(1, HC)
    rb_b12 = rb_b1.reshape(1, HC)
    rb_b22 = rb_b2.reshape(1, HC)
    b_lin2 = b_lin.reshape(1, HC)
    a0b12 = ra0_b1.reshape(1, HC)
    a0b22 = ra0_b2.reshape(1, HC)
    a1b12 = ra1_b1.reshape(1, HC)
    a1b22 = ra1_b2.reshape(1, HC)

    nA = E // BEA
    xji, xkd = pl.pallas_call(
        _tca_body,
        grid=(nA,),
        in_specs=[
            pl.BlockSpec((BEA, HC), lambda i: (i, 0)),
            pl.BlockSpec((BEA, 6), lambda i: (i, 0)),
            _rep((HC, HC)), _rep((1, HC)), _rep((HC, HC)), _rep((1, HC)),
            _rep((6, 8)), _rep((8, HC)), _rep((HC, INT)),
        ],
        out_specs=[
            pl.BlockSpec((BEA, HC), lambda i: (i, 0)),
            pl.BlockSpec((BEA, HC), lambda i: (i, 0)),
        ],
        out_shape=[
            jax.ShapeDtypeStruct((E, HC), f32),
            jax.ShapeDtypeStruct((E, HC), f32),
        ],
        compiler_params=pltpu.CompilerParams(
            dimension_semantics=("parallel",)),
    )(x1, rbf0, W_ji, b_ji2, W_kj, b_kj2, W_rbf1, W_rbf2, W_down)

    nB = T // BTB
    m = pl.pallas_call(
        _tcb_body,
        grid=(nB,),
        in_specs=[
            pl.BlockSpec((BTB, 18), lambda i: (i, 0)),
            pl.BlockSpec((BTB, 54), lambda i: (i, 0)),
            _rep((18, 8)), _rep((8, INT)), _rep((54, 8)), _rep((8, INT)),
        ],
        out_specs=pl.BlockSpec((BTB, HC), lambda i: (i, 0)),
        out_shape=jax.ShapeDtypeStruct((T, HC), f32),
        compiler_params=pltpu.CompilerParams(
            dimension_semantics=("parallel",)),
    )(sbf, t, W_sbf1, W_sbf2, W_t1, W_t2)

    ji4 = idx_ji.reshape(NBLOCK, BLK // 128, 128)
    enc = pl.pallas_call(
        _tcp_body,
        grid=(NBLOCK,),
        in_specs=[pl.BlockSpec((1, BLK // 128, 128), lambda i: (i, 0, 0))],
        out_specs=pl.BlockSpec((1, NRANGE, BLK // 128 + 1, 128),
                               lambda i: (i, 0, 0, 0)),
        out_shape=jax.ShapeDtypeStruct((NBLOCK, NRANGE, BLK // 128 + 1, 128),
                                       jnp.int32),
        compiler_params=pltpu.CompilerParams(
            dimension_semantics=("parallel",)),
    )(ji4)
    enc_flat = enc.reshape(NBLOCK * NRANGE * ENCW)

    zrows = jnp.zeros((SUBROWS, INT), f32)
    ypad = _sc_scatter(xkd, m, idx_kj, idx_ji, enc_flat, zrows)
    y = ypad[:E]

    e1, e2 = pl.pallas_call(
        _tcc_body,
        grid=(nA,),
        in_specs=[
            pl.BlockSpec((BEA, INT), lambda i: (i, 0)),
            pl.BlockSpec((BEA, HC), lambda i: (i, 0)),
            pl.BlockSpec((BEA, HC), lambda i: (i, 0)),
            pl.BlockSpec((BEA, 6), lambda i: (i, 0)),
            _rep((INT, HC)),
            _rep((HC, HC)), _rep((1, HC)), _rep((HC, HC)), _rep((1, HC)),
            _rep((HC, HC)), _rep((1, HC)),
            _rep((HC, HC)), _rep((1, HC)), _rep((HC, HC)), _rep((1, HC)),
            _rep((HC, HC)), _rep((1, HC)), _rep((HC, HC)), _rep((1, HC)),
            _rep((6, HC)),
        ],
        out_specs=[
            pl.BlockSpec((BEA, HC), lambda i: (i, 0)),
            pl.BlockSpec((BEA, HC), lambda i: (i, 0)),
        ],
        out_shape=[
            jax.ShapeDtypeStruct((E, HC), f32),
            jax.ShapeDtypeStruct((E, HC), f32),
        ],
        compiler_params=pltpu.CompilerParams(
            dimension_semantics=("parallel",)),
    )(y, xji, x1, rbf0, W_up,
      rb_W1, rb_b12, rb_W2, rb_b22,
      W_lin, b_lin2,
      ra0_W1, a0b12, ra0_W2, a0b22,
      ra1_W1, a1b12, ra1_W2, a1b22,
      W_rbf)

    return (e1, e2)


# speedup vs baseline: 1.0550x; 1.0011x over previous
import functools

import jax
import jax.numpy as jnp
from jax import lax
from jax.experimental import pallas as pl
from jax.experimental.pallas import tpu as pltpu
from jax.experimental.pallas import tpu_sc as plsc

E = 160000
T = 640000
HC = 128
INT = 64

NCORE = 2
NSUB = 16
NRANGE = 14
RANGE = 11520
EPAD = NRANGE * RANGE
BLK = 2560
BROWS = BLK // 128
NBLOCK = T // BLK
NB = 128
TRASH0 = BLK + NB
CAP = BLK + 192
POSW = (BROWS + 1) * 128
SUBROWS = RANGE // NSUB

BEA = 4000
BTB = 8000


def _silu(x):
    return x * jax.nn.sigmoid(x)


def _tca_body(x1_ref, rbf0_ref, Wji_ref, bji_ref, Wkj_ref, bkj_ref,
              Wr1_ref, Wr2_ref, Wd_ref, xji_ref, xkd_ref):
    x1 = x1_ref[...]
    xji_ref[...] = _silu(x1 @ Wji_ref[...] + bji_ref[...])
    rbf = (rbf0_ref[...] @ Wr1_ref[...]) @ Wr2_ref[...]
    xk = _silu(x1 @ Wkj_ref[...] + bkj_ref[...]) * rbf
    xkd = _silu(xk @ Wd_ref[...])
    xkd_ref[...] = jnp.concatenate(
        [xkd, jnp.zeros((xkd.shape[0], HC - INT), xkd.dtype)], axis=1)


def _tcb_body(sbf_ref, t_ref, Ws1_ref, Ws2_ref, Wt1_ref, Wt2_ref, m_ref):
    A = Ws1_ref[...] @ Ws2_ref[...]
    B = Wt1_ref[...] @ Wt2_ref[...]
    mm = (sbf_ref[...] @ A) * (t_ref[...] @ B)
    m_ref[...] = jnp.concatenate(
        [mm, jnp.zeros((mm.shape[0], HC - INT), mm.dtype)], axis=1)


def _tcp_body(ji_ref, kj_ref, pos_ref, pack_ref):
    f32 = jnp.float32
    jib = ji_ref[0]
    kjb = kj_ref[0]
    owner_off = (pl.program_id(0) % NSUB) * CAP
    lane16 = lax.broadcasted_iota(jnp.int32, (BROWS, 128), 1) % 16
    tl = (lax.broadcasted_iota(jnp.int32, (BROWS, 128), 0) * 128
          + lax.broadcasted_iota(jnp.int32, (BROWS, 128), 1))
    U = (lax.broadcasted_iota(jnp.int32, (128, 128), 0)
         <= lax.broadcasted_iota(jnp.int32, (128, 128), 1)).astype(f32)
    Ls = (lax.broadcasted_iota(jnp.int32, (BROWS, BROWS), 1)
          < lax.broadcasted_iota(jnp.int32, (BROWS, BROWS), 0)).astype(f32)
    for r in range(NRANGE):
        lo = r * RANGE
        mk = (jib >= lo) & (jib < lo + RANGE)
        mf = mk.astype(f32)
        p = jax.lax.dot(mf, U, preferred_element_type=f32)
        offs = jax.lax.dot(Ls, p, preferred_element_type=f32)[:, 127:128]
        pos = p + offs
        enc = jnp.where(mk, pos.astype(jnp.int32) - 1, TRASH0 + lane16)
        cnt = pos[BROWS - 1:BROWS, 127:128].astype(jnp.int32)
        cnt_row = jnp.broadcast_to(cnt, (1, 128))
        pos_ref[0, r] = jnp.concatenate([enc + owner_off, cnt_row], axis=0)
    pack_ref[0] = tl + kjb * 0


def _tcc_body(y_ref, xji_ref, x1_ref, rbf0_ref, Wup_ref,
              rbW1_ref, rbb1_ref, rbW2_ref, rbb2_ref,
              Wlin_ref, blin_ref,
              a0W1_ref, a0b1_ref, a0W2_ref, a0b2_ref,
              a1W1_ref, a1b1_ref, a1W2_ref, a1b2_ref,
              Wrbf_ref, e1_ref, e2_ref):
    e1 = xji_ref[...] + _silu(y_ref[...] @ Wup_ref[...])
    e1 = e1 + _silu(_silu(e1 @ rbW1_ref[...] + rbb1_ref[...])
                    @ rbW2_ref[...] + rbb2_ref[...])
    e1 = _silu(e1 @ Wlin_ref[...] + blin_ref[...]) + x1_ref[...]
    e1 = e1 + _silu(_silu(e1 @ a0W1_ref[...] + a0b1_ref[...])
                    @ a0W2_ref[...] + a0b2_ref[...])
    e1 = e1 + _silu(_silu(e1 @ a1W1_ref[...] + a1b1_ref[...])
                    @ a1W2_ref[...] + a1b2_ref[...])
    e1_ref[...] = e1
    e2_ref[...] = (rbf0_ref[...] @ Wrbf_ref[...]) * e1


def _sc_body(tbl, m, pos, kj, ji, zrows, y,
             acc, sp_tid, sp_kj, sp_d,
             pos_v, cnt_v, kj_v, ji_v, tids_v,
             tid_loc, kj_loc, d_loc, dstb_v, mrows_v, trows_v, prod_v,
             sem_s1, sem_s2):
    cid = lax.axis_index("c")
    sid = lax.axis_index("s")
    row0 = sid * SUBROWS
    sid_cap = sid * CAP
    lanes = lax.iota(jnp.int32, 16)

    pltpu.sync_copy(zrows, acc.at[pl.ds(row0, SUBROWS)])
    plsc.subcore_barrier()

    nblk = jnp.where(sid < NBLOCK - (NBLOCK // NSUB) * NSUB,
                     NBLOCK // NSUB + 1, NBLOCK // NSUB)

    for ri in range(NRANGE // NCORE):
        r = cid * (NRANGE // NCORE) + ri
        lo = r * RANGE

        def block_body(k, _, r=r, lo=lo):
            b = sid + k * NSUB
            tbase = b * BLK
            pbase = (b * NRANGE + r) * POSW
            pltpu.sync_copy(pos.at[pl.ds(pbase, BLK)], pos_v)
            pltpu.sync_copy(pos.at[pl.ds(pbase + BLK, 16)], cnt_v)
            pltpu.sync_copy(kj.at[pl.ds(tbase, BLK)], kj_v)
            pltpu.sync_copy(ji.at[pl.ds(tbase, BLK)], ji_v)

            def chunk_body(i, _):
                o = i * 16
                ji_v[pl.ds(o, 16)] = ji_v[pl.ds(o, 16)] - lo
                tids_v[pl.ds(o, 16)] = (tbase + o) + lanes
                return 0

            lax.fori_loop(0, BLK // 16, chunk_body, 0)

            pltpu.sync_copy(tids_v, sp_tid.at[pos_v])
            pltpu.sync_copy(kj_v, sp_kj.at[pos_v])
            pltpu.sync_copy(ji_v, sp_d.at[pos_v])
            pltpu.sync_copy(sp_tid.at[pl.ds(sid_cap, CAP)], tid_loc)
            pltpu.sync_copy(sp_kj.at[pl.ds(sid_cap, CAP)], kj_loc)
            pltpu.sync_copy(sp_d.at[pl.ds(sid_cap, CAP)], d_loc)

            cnt = cnt_v[pl.ds(0, 16)][0]
            cnt = jnp.minimum(jnp.maximum(cnt, 0), BLK)

            for k2 in range(NB // 16):
                tid_loc[pl.ds(cnt + k2 * 16, 16)] = jnp.zeros((16,),
                                                              jnp.int32)
                kj_loc[pl.ds(cnt + k2 * 16, 16)] = jnp.zeros((16,),
                                                             jnp.int32)
                d_loc[pl.ds(cnt + k2 * 16, 16)] = jnp.full((16,), RANGE,
                                                           jnp.int32)
            nb = (cnt + (NB - 1)) >> 7

            def batch_body(bb, _):
                off = bb * NB
                for k3 in range(NB // 16):
                    d = d_loc[pl.ds(off + k3 * 16, 16)]
                    dstb_v[pl.ds(k3 * 16, 16)] = jnp.minimum(
                        jnp.maximum(d, 0), jnp.int32(RANGE))
                g1 = pltpu.async_copy(m.at[tid_loc.at[pl.ds(off, NB)]],
                                      mrows_v, sem_s1)
                g2 = pltpu.async_copy(tbl.at[kj_loc.at[pl.ds(off, NB)]],
                                      trows_v, sem_s2)
                g1.wait()
                g2.wait()

                def mul_row(i2, _):
                    for jj in range(INT // 16):
                        tc = trows_v[i2, pl.ds(jj * 16, 16)]
                        mc = mrows_v[i2, pl.ds(jj * 16, 16)]
                        prod_v[i2, pl.ds(jj * 16, 16)] = tc * mc
                    return 0

                lax.fori_loop(0, NB, mul_row, 0)
                pltpu.sync_copy(prod_v, acc.at[dstb_v], add=True)
                return 0

            lax.fori_loop(0, nb, batch_body, 0)
            return 0

        lax.fori_loop(0, nblk, block_body, 0)

        plsc.subcore_barrier()
        pltpu.sync_copy(acc.at[pl.ds(row0, SUBROWS)],
                        y.at[pl.ds(lo + row0, SUBROWS)])
        if ri != NRANGE // NCORE - 1:
            pltpu.sync_copy(zrows, acc.at[pl.ds(row0, SUBROWS)])
        plsc.subcore_barrier()


_sc_scatter = functools.partial(
    pl.kernel,
    out_type=jax.ShapeDtypeStruct((EPAD, INT), jnp.float32),
    mesh=plsc.VectorSubcoreMesh(core_axis_name="c", subcore_axis_name="s"),
    scratch_types=[
        pltpu.VMEM_SHARED((RANGE + 8, INT), jnp.float32),
        pltpu.VMEM_SHARED((NSUB * CAP,), jnp.int32),
        pltpu.VMEM_SHARED((NSUB * CAP,), jnp.int32),
        pltpu.VMEM_SHARED((NSUB * CAP,), jnp.int32),
        pltpu.VMEM((BLK,), jnp.int32),
        pltpu.VMEM((16,), jnp.int32),
        pltpu.VMEM((BLK,), jnp.int32),
        pltpu.VMEM((BLK,), jnp.int32),
        pltpu.VMEM((BLK,), jnp.int32),
        pltpu.VMEM((CAP,), jnp.int32),
        pltpu.VMEM((CAP,), jnp.int32),
        pltpu.VMEM((CAP,), jnp.int32),
        pltpu.VMEM((NB,), jnp.int32),
        pltpu.VMEM((NB, HC), jnp.float32),
        pltpu.VMEM((NB, HC), jnp.float32),
        pltpu.VMEM((NB, INT), jnp.float32),
        pltpu.SemaphoreType.DMA,
        pltpu.SemaphoreType.DMA,
    ],
)(_sc_body)


def _rep(shape):
    nd = len(shape)
    return pl.BlockSpec(shape, lambda i, _nd=nd: (0,) * _nd)


def kernel(x1, x2, rbf0, sbf, t, idx_kj, idx_ji,
           W_rbf1, W_rbf2, W_sbf1, W_sbf2, W_t1, W_t2, W_rbf,
           W_kj, b_kj, W_ji, b_ji, W_down, W_up,
           rb_W1, rb_b1, rb_W2, rb_b2,
           W_lin, b_lin,
           ra0_W1, ra0_b1, ra0_W2, ra0_b2,
           ra1_W1, ra1_b1, ra1_W2, ra1_b2):
    f32 = jnp.float32
    idx_kj = idx_kj.astype(jnp.int32)
    idx_ji = idx_ji.astype(jnp.int32)
    b_kj2 = b_kj.reshape(1, HC)
    b_ji2 = b_ji.reshape(1, HC)
    rb_b12 = rb_b1.reshape(1, HC)
    rb_b22 = rb_b2.reshape(1, HC)
    b_lin2 = b_lin.reshape(1, HC)
    a0b12 = ra0_b1.reshape(1, HC)
    a0b22 = ra0_b2.reshape(1, HC)
    a1b12 = ra1_b1.reshape(1, HC)
    a1b22 = ra1_b2.reshape(1, HC)

    nA = E // BEA
    xji, xkd = pl.pallas_call(
        _tca_body,
        grid=(nA,),
        in_specs=[
            pl.BlockSpec((BEA, HC), lambda i: (i, 0)),
            pl.BlockSpec((BEA, 6), lambda i: (i, 0)),
            _rep((HC, HC)), _rep((1, HC)), _rep((HC, HC)), _rep((1, HC)),
            _rep((6, 8)), _rep((8, HC)), _rep((HC, INT)),
        ],
        out_specs=[
            pl.BlockSpec((BEA, HC), lambda i: (i, 0)),
            pl.BlockSpec((BEA, HC), lambda i: (i, 0)),
        ],
        out_shape=[
            jax.ShapeDtypeStruct((E, HC), f32),
            jax.ShapeDtypeStruct((E, HC), f32),
        ],
        compiler_params=pltpu.CompilerParams(
            dimension_semantics=("parallel",)),
    )(x1, rbf0, W_ji, b_ji2, W_kj, b_kj2, W_rbf1, W_rbf2, W_down)

    nB = T // BTB
    m = pl.pallas_call(
        _tcb_body,
        grid=(nB,),
        in_specs=[
            pl.BlockSpec((BTB, 18), lambda i: (i, 0)),
            pl.BlockSpec((BTB, 54), lambda i: (i, 0)),
            _rep((18, 8)), _rep((8, INT)), _rep((54, 8)), _rep((8, INT)),
        ],
        out_specs=pl.BlockSpec((BTB, HC), lambda i: (i, 0)),
        out_shape=jax.ShapeDtypeStruct((T, HC), f32),
        compiler_params=pltpu.CompilerParams(
            dimension_semantics=("parallel",)),
    )(sbf, t, W_sbf1, W_sbf2, W_t1, W_t2)

    ji4 = idx_ji.reshape(NBLOCK, BROWS, 128)
    kj4 = idx_kj.reshape(NBLOCK, BROWS, 128)
    posr, packr = pl.pallas_call(
        _tcp_body,
        grid=(NBLOCK,),
        in_specs=[
            pl.BlockSpec((1, BROWS, 128), lambda i: (i, 0, 0)),
            pl.BlockSpec((1, BROWS, 128), lambda i: (i, 0, 0)),
        ],
        out_specs=[
            pl.BlockSpec((1, NRANGE, BROWS + 1, 128), lambda i: (i, 0, 0, 0)),
            pl.BlockSpec((1, BROWS, 128), lambda i: (i, 0, 0)),
        ],
        out_shape=[
            jax.ShapeDtypeStruct((NBLOCK, NRANGE, BROWS + 1, 128), jnp.int32),
            jax.ShapeDtypeStruct((NBLOCK, BROWS, 128), jnp.int32),
        ],
        compiler_params=pltpu.CompilerParams(
            dimension_semantics=("parallel",)),
    )(ji4, kj4)
    pos_flat = posr.reshape(NBLOCK * NRANGE * POSW)
    pack_flat = packr.reshape(T)

    zrows = jnp.zeros((SUBROWS, INT), f32)
    ypad = _sc_scatter(xkd, m, pos_flat, idx_kj, idx_ji, zrows)
    y = ypad[:E]

    e1, e2 = pl.pallas_call(
        _tcc_body,
        grid=(nA,),
        in_specs=[
            pl.BlockSpec((BEA, INT), lambda i: (i, 0)),
            pl.BlockSpec((BEA, HC), lambda i: (i, 0)),
            pl.BlockSpec((BEA, HC), lambda i: (i, 0)),
            pl.BlockSpec((BEA, 6), lambda i: (i, 0)),
            _rep((INT, HC)),
            _rep((HC, HC)), _rep((1, HC)), _rep((HC, HC)), _rep((1, HC)),
            _rep((HC, HC)), _rep((1, HC)),
            _rep((HC, HC)), _rep((1, HC)), _rep((HC, HC)), _rep((1, HC)),
            _rep((HC, HC)), _rep((1, HC)), _rep((HC, HC)), _rep((1, HC)),
            _rep((6, HC)),
        ],
        out_specs=[
            pl.BlockSpec((BEA, HC), lambda i: (i, 0)),
            pl.BlockSpec((BEA, HC), lambda i: (i, 0)),
        ],
        out_shape=[
            jax.ShapeDtypeStruct((E, HC), f32),
            jax.ShapeDtypeStruct((E, HC), f32),
        ],
        compiler_params=pltpu.CompilerParams(
            dimension_semantics=("parallel",)),
    )(y, xji, x1, rbf0, W_up,
      rb_W1, rb_b12, rb_W2, rb_b22,
      W_lin, b_lin2,
      ra0_W1, a0b12, ra0_W2, a0b22,
      ra1_W1, a1b12, ra1_W2, a1b22,
      W_rbf)

    return (e1, e2)
